# baseline (speedup 1.0000x reference)
import jax
import jax.numpy as jnp
from jax.experimental import pallas as pl
from jax.experimental.pallas import tpu as pltpu


def _router_block(x_ref, w_ref, b_ref, o_ref):
    logits = jnp.dot(x_ref[...], w_ref[...], preferred_element_type=jnp.float32)
    logits = logits + b_ref[...]
    m = jnp.max(logits, axis=-1, keepdims=True)
    e = jnp.exp(logits - m)
    o_ref[...] = e / jnp.sum(e, axis=-1, keepdims=True)


def kernel(x, W, b):
    N, D = x.shape
    E = W.shape[0]
    BLOCK = 1024
    Wt = W.T
    b2 = b.reshape(1, E)
    out = pl.pallas_call(
        _router_block,
        grid=(N // BLOCK,),
        in_specs=[
            pl.BlockSpec((BLOCK, D), lambda i: (i, 0)),
            pl.BlockSpec((D, E), lambda i: (0, 0)),
            pl.BlockSpec((1, E), lambda i: (0, 0)),
        ],
        out_specs=pl.BlockSpec((BLOCK, E), lambda i: (i, 0)),
        out_shape=jax.ShapeDtypeStruct((N, E), jnp.float32),
        compiler_params=pltpu.CompilerParams(
            dimension_semantics=("parallel",),
        ),
    )(x, Wt, b2)
    return out


# speedup vs baseline: 1.1523x; 1.1523x over previous
import jax
import jax.numpy as jnp
from jax.experimental import pallas as pl
from jax.experimental.pallas import tpu as pltpu

N = 32768
D = 768
E = 8
BLK = 2048
NBLK = N // BLK
NBUF = 4


def _body(x_hbm, w_ref, b_ref, o_ref, *scr):
    xbufs = scr[:NBUF]
    sems = scr[NBUF:]

    def src(i):
        return x_hbm.at[pl.ds(i * BLK, BLK), :]

    for i in range(min(NBUF, NBLK)):
        pltpu.make_async_copy(src(i), xbufs[i], sems[i]).start()

    for i in range(NBLK):
        bi = i % NBUF
        pltpu.make_async_copy(src(i), xbufs[bi], sems[bi]).wait()
        logits = jnp.dot(
            xbufs[bi][...], w_ref[...], preferred_element_type=jnp.float32
        ) + b_ref[...]
        m = jnp.max(logits, axis=-1, keepdims=True)
        ex = jnp.exp(logits - m)
        o_ref[pl.ds(i * BLK, BLK), :] = ex / jnp.sum(ex, axis=-1, keepdims=True)
        if i + NBUF < NBLK:
            pltpu.make_async_copy(src(i + NBUF), xbufs[bi], sems[bi]).start()


def kernel(x, W, b):
    Wt = W.T
    b2 = b.reshape(1, E)
    out = pl.pallas_call(
        _body,
        in_specs=[
            pl.BlockSpec(memory_space=pltpu.MemorySpace.HBM),
            pl.BlockSpec(memory_space=pltpu.VMEM),
            pl.BlockSpec(memory_space=pltpu.VMEM),
        ],
        out_specs=pl.BlockSpec(memory_space=pltpu.VMEM),
        out_shape=jax.ShapeDtypeStruct((N, E), jnp.float32),
        scratch_shapes=(
            [pltpu.VMEM((BLK, D), jnp.float32) for _ in range(NBUF)]
            + [pltpu.SemaphoreType.DMA for _ in range(NBUF)]
        ),
    )(x, Wt, b2)
    return out
